# Initial kernel scaffold; baseline (speedup 1.0000x reference)
import jax
import jax.numpy as jnp
from jax.experimental import pallas as pl


def kernel(x, Wg, bg, Wn, bn, W1, b1, W2, b2, eps):
    raise NotImplementedError("write your pallas kernel here")


# speedup vs baseline: 1.3058x; 1.3058x over previous
import functools

import jax
import jax.numpy as jnp
from jax.experimental import pallas as pl
from jax.experimental.pallas import tpu as pltpu


def _router_body(x_ref, wg_ref, bg_ref, wn_ref, bn_ref, eps_ref, g_ref):
    xb = x_ref[...]
    lg = jnp.dot(xb, wg_ref[...], preferred_element_type=jnp.float32) + bg_ref[...]
    nz = jnp.dot(xb, wn_ref[...], preferred_element_type=jnp.float32) + bn_ref[...]
    sp = jnp.maximum(nz, 0.0) + jnp.log1p(jnp.exp(-jnp.abs(nz)))
    nl = lg + eps_ref[...] * sp
    e = nl.shape[-1]
    m1 = jnp.max(nl, axis=-1, keepdims=True)
    ii = jax.lax.broadcasted_iota(jnp.int32, nl.shape, 1)
    fmi = jnp.min(jnp.where(nl == m1, ii, e), axis=-1, keepdims=True)
    m2 = jnp.max(jnp.where(ii == fmi, -jnp.inf, nl), axis=-1, keepdims=True)
    sel = (ii == fmi) | (nl >= m2)
    z = jnp.where(sel, jnp.exp(nl - m1), 0.0)
    g_ref[...] = z / jnp.sum(z, axis=-1, keepdims=True)


def _expert_body(x_ref, g_ref, w1_ref, b1_ref, w2_ref, b2_ref, o_ref):
    e = pl.program_id(1)
    f = pl.program_id(2)

    @pl.when((e == 0) & (f == 0))
    def _init():
        o_ref[...] = jnp.dot(g_ref[...], b2_ref[...],
                             preferred_element_type=jnp.float32)

    xb = x_ref[...]
    h = jnp.maximum(
        jnp.dot(xb, w1_ref[0], preferred_element_type=jnp.float32) + b1_ref[0], 0.0)
    p = jnp.dot(h, w2_ref[0], preferred_element_type=jnp.float32)
    lane = jax.lax.broadcasted_iota(jnp.int32, g_ref.shape, 1)
    g = jnp.sum(jnp.where(lane == e, g_ref[...], 0.0), axis=-1, keepdims=True)
    o_ref[...] += p * g


@functools.partial(jax.jit, static_argnames=())
def kernel(x, Wg, bg, Wn, bn, W1, b1, W2, b2, eps):
    B, S, D = x.shape
    E = Wg.shape[1]
    FF = W1.shape[2]
    N = B * S
    x2 = x.reshape(N, D)
    eps2 = eps.reshape(N, E)

    bt_r = min(2048, N)
    gates = pl.pallas_call(
        _router_body,
        grid=(N // bt_r,),
        in_specs=[
            pl.BlockSpec((bt_r, D), lambda t: (t, 0)),
            pl.BlockSpec((D, E), lambda t: (0, 0)),
            pl.BlockSpec((1, E), lambda t: (0, 0)),
            pl.BlockSpec((D, E), lambda t: (0, 0)),
            pl.BlockSpec((1, E), lambda t: (0, 0)),
            pl.BlockSpec((bt_r, E), lambda t: (t, 0)),
        ],
        out_specs=pl.BlockSpec((bt_r, E), lambda t: (t, 0)),
        out_shape=jax.ShapeDtypeStruct((N, E), jnp.float32),
        compiler_params=pltpu.CompilerParams(
            dimension_semantics=("arbitrary",)),
    )(x2, Wg, bg.reshape(1, E), Wn, bn.reshape(1, E), eps2)

    bt = min(2048, N)
    bf = min(1024, FF)
    out = pl.pallas_call(
        _expert_body,
        grid=(N // bt, E, FF // bf),
        in_specs=[
            pl.BlockSpec((bt, D), lambda t, e, f: (t, 0)),
            pl.BlockSpec((bt, E), lambda t, e, f: (t, 0)),
            pl.BlockSpec((1, D, bf), lambda t, e, f: (e, 0, f)),
            pl.BlockSpec((1, 1, bf), lambda t, e, f: (e, 0, f)),
            pl.BlockSpec((1, bf, D), lambda t, e, f: (e, f, 0)),
            pl.BlockSpec((E, D), lambda t, e, f: (0, 0)),
        ],
        out_specs=pl.BlockSpec((bt, D), lambda t, e, f: (t, 0)),
        out_shape=jax.ShapeDtypeStruct((N, D), jnp.float32),
        compiler_params=pltpu.CompilerParams(
            dimension_semantics=("parallel", "arbitrary", "arbitrary"),
            vmem_limit_bytes=100 * 1024 * 1024),
    )(x2, gates, W1, b1.reshape(E, 1, FF), W2, b2)

    return out.reshape(B, S, D)
